# Initial kernel scaffold; baseline (speedup 1.0000x reference)
import jax
import jax.numpy as jnp
from jax.experimental import pallas as pl


def kernel(outputs, targets):
    raise NotImplementedError("write your pallas kernel here")


# speedup vs baseline: 2.0050x; 2.0050x over previous
import functools

import jax
import jax.numpy as jnp
from jax import lax
from jax.experimental import pallas as pl
from jax.experimental.pallas import tpu as pltpu
from jax.experimental.pallas import tpu_sc as plsc

N = 262144
R, C = 2048, 128
RADIX = 1024
KEY_POS = 0x3F800000
KEY_BASE = 0x3F7FFFFF

SCW = 16
CH = N // SCW
S = 4
VW = SCW * S
SUB = CH // S


def _weights(q, npos, nneg):
  nh = 4 * npos
  nh = jnp.where(nh == 0, 100, nh)
  itv = nh // 5
  ld = nneg // 5
  p = q - 200
  mult = jnp.zeros_like(q)
  for i in range(5):
    st = i * ld
    mult = mult + ((p >= st) & (p < st + itv)).astype(jnp.int32)
  return jnp.where((q >= 200) & (q < nneg), mult, 0)


def _tc_pre(v_ref, t_ref, key_ref, cex_ref, possum_ref, npos_ref, nneg_ref):
  v = v_ref[...]
  t = t_ref[...]
  neg = t < 1
  bits = lax.bitcast_convert_type(v, jnp.int32)
  key_ref[...] = jnp.where(neg, KEY_BASE - bits, KEY_POS)

  m = neg.astype(jnp.float32)
  iu = lax.broadcasted_iota(jnp.int32, (C, C), 0)
  il = lax.broadcasted_iota(jnp.int32, (C, C), 1)
  U = (iu < il).astype(jnp.float32)
  rowex = jnp.dot(m, U, preferred_element_type=jnp.float32)
  rowsum = jnp.sum(m, axis=1, keepdims=True)
  x = rowsum
  s = 1
  while s < R:
    x = x + jnp.pad(x, ((s, 0), (0, 0)))[:R]
    s *= 2
  rowpref = x - rowsum
  cex_ref[...] = (rowex + rowpref).astype(jnp.int32)

  logp = jnp.maximum(jnp.log(v), -100.0)
  possum_ref[...] = jnp.reshape(jnp.sum(jnp.where(neg, 0.0, -logp)), (1, 1))
  npos_ref[...] = jnp.reshape(jnp.sum(jnp.where(neg, 0, 1)), (1, 1))
  nneg_ref[...] = jnp.reshape(jnp.sum(jnp.where(neg, 1, 0)), (1, 1))


def _tc_post(t_ref, wq_ref, sg_ref, possum_ref, npos_ref, nneg_ref, out_ref):
  t = t_ref[...]
  neg = t < 1
  wq = wq_ref[...].astype(jnp.float32)
  g = lax.bitcast_convert_type(KEY_BASE - sg_ref[...], jnp.float32)
  l1 = jnp.maximum(jnp.log(1.0 - g), -100.0)
  numer = jnp.sum(jnp.where(neg, wq * (-l1), 0.0))

  npos = npos_ref[0, 0]
  nneg = nneg_ref[0, 0]
  q = lax.broadcasted_iota(jnp.int32, (R, C), 0) * C + lax.broadcasted_iota(
      jnp.int32, (R, C), 1)
  denom = jnp.sum(_weights(q, npos, nneg).astype(jnp.float32))
  pos_loss = jnp.where(
      npos > 0,
      possum_ref[0, 0] / jnp.maximum(npos, 1).astype(jnp.float32),
      jnp.float32(0.0),
  )
  out_ref[...] = jnp.reshape(pos_loss + numer / denom, (1, 1))


def _sc_fused():
  mesh = plsc.VectorSubcoreMesh(
      core_axis_name="c", subcore_axis_name="s", num_cores=1)

  out_type = [
      jax.ShapeDtypeStruct((N,), jnp.int32),
      jax.ShapeDtypeStruct((N,), jnp.int32),
  ]
  scratch = [
      pltpu.VMEM((CH,), jnp.int32),
      pltpu.VMEM((CH,), jnp.int32),
      pltpu.VMEM((RADIX,), jnp.int32),
      pltpu.VMEM((RADIX,), jnp.int32),
      pltpu.VMEM((RADIX,), jnp.int32),
      pltpu.VMEM((RADIX,), jnp.int32),
      pltpu.VMEM((VW, 128), jnp.int32),
      pltpu.VMEM((CH,), jnp.int32),
      pltpu.VMEM((CH,), jnp.int32),
      pltpu.VMEM((CH,), jnp.int32),
      pltpu.VMEM((16,), jnp.int32),
      pltpu.VMEM_SHARED((VW, RADIX), jnp.int32),
      pltpu.VMEM_SHARED((N,), jnp.int32),
      pltpu.SemaphoreType.DMA,
  ]

  def body(keyin, cexin, scal, wqout, sgout, keych, paych, o0, o1, o2, o3,
           strip, stageidx, stagekey, stagepay, scalv, grid, pay, sem):
    offs_l = [o0, o1, o2, o3]
    wid = lax.axis_index("s")
    base = wid * CH
    widS = wid * S
    lanes = lax.iota(jnp.int32, 16)
    zero16 = jnp.zeros((16,), jnp.int32)

    pltpu.sync_copy(scal, scalv)
    sv = scalv[...]
    nposb = sv[0]
    nnegb = sv[1]
    nh = 4 * nposb
    nh = jnp.where(nh == 0, 100, nh)
    itv = nh // 5
    ld = nnegb // 5

    def run_pass(pass_idx):
      shift = 10 * pass_idx
      first = pass_idx == 0

      if first:
        pltpu.sync_copy(keyin.at[pl.ds(base, CH)], keych)
      else:
        pltpu.sync_copy(pay.at[pl.ds(base, CH)], paych)
        pltpu.async_copy(keyin.at[paych], keych, sem).wait()

      def zbody(i, _):
        for c in range(S):
          offs_l[c][pl.ds(i * 16, 16)] = zero16
        return 0

      lax.fori_loop(0, RADIX // 16, zbody, 0)

      def ha(r, _):
        for c in range(S):
          for u in range(2):
            j = r * 2 + u
            k = keych[pl.ds(c * SUB + j * 16, 16)]
            d = (k >> shift) & (RADIX - 1)
            cnt, lastm = plsc.scan_count(d)
            plsc.addupdate_scatter(offs_l[c], [d], cnt, mask=lastm)
        return 0

      lax.fori_loop(0, SUB // 32, ha, 0)

      for c in range(S):
        pltpu.sync_copy(offs_l[c], grid.at[widS + c])
      plsc.subcore_barrier()

      def hb(jb, carry):
        pltpu.sync_copy(grid.at[:, pl.ds(jb * 128, 128)], strip)

        def hbj(jj, carry):
          sl = pl.ds(jj * 16, 16)
          tot = jnp.zeros((16,), jnp.int32)
          for vw in range(VW):
            tot = tot + strip[vw, sl]
          inc = plsc.cumsum(tot)
          exc = inc - tot + carry
          part = jnp.zeros((16,), jnp.int32)
          for vw in range(VW):
            part = part + jnp.where(vw < widS, strip[vw, sl], 0)
          for c in range(S):
            offs_l[c][pl.ds(jb * 128 + jj * 16, 16)] = exc + part
            if c < S - 1:
              part = part + strip[widS + c, sl]
          return carry + jnp.sum(tot)

        return lax.fori_loop(0, 8, hbj, carry)

      lax.fori_loop(0, RADIX // 128, hb, jnp.int32(0))

      def hc(r, _):
        for c in range(S):
          for u in range(2):
            j = r * 2 + u
            off = c * SUB + j * 16
            k = keych[pl.ds(off, 16)]
            if first:
              p = base + off + lanes
            else:
              p = paych[pl.ds(off, 16)]
            d = (k >> shift) & (RADIX - 1)
            cnt, lastm = plsc.scan_count(d)
            bs = plsc.load_gather(offs_l[c], [d])
            pos = bs + cnt - 1
            plsc.addupdate_scatter(offs_l[c], [d], cnt, mask=lastm)
            stageidx[pl.ds(off, 16)] = pos
            stagepay[pl.ds(off, 16)] = p
        return 0

      lax.fori_loop(0, SUB // 32, hc, 0)

      pltpu.async_copy(stagepay, pay.at[stageidx], sem).wait()
      plsc.subcore_barrier()

    run_pass(0)
    run_pass(1)
    run_pass(2)

    pltpu.sync_copy(cexin.at[pl.ds(base, CH)], keych)
    pltpu.async_copy(pay.at[keych], stagepay, sem).wait()
    pltpu.async_copy(keyin.at[stagepay], stagekey, sem).wait()
    pltpu.sync_copy(stagekey, sgout.at[pl.ds(base, CH)])
    pltpu.sync_copy(pay.at[pl.ds(base, CH)], paych)

    def hw(r, _):
      for u in range(8):
        off = r * 128 + u * 16
        q = base + off + lanes
        pp = q - 200
        mult = jnp.zeros((16,), jnp.int32)
        for i in range(5):
          st = i * ld
          mult = mult + ((pp >= st) & (pp < st + itv)).astype(jnp.int32)
        wv = jnp.where((q >= 200) & (q < nnegb), mult, 0)
        stagekey[pl.ds(off, 16)] = wv
      return 0

    lax.fori_loop(0, CH // 128, hw, 0)
    plsc.subcore_barrier()
    pltpu.async_copy(stagekey, pay.at[paych], sem).wait()
    plsc.subcore_barrier()
    pltpu.sync_copy(pay.at[pl.ds(base, CH)], wqout.at[pl.ds(base, CH)])

  return pl.kernel(
      body,
      out_type=out_type,
      mesh=mesh,
      scratch_types=scratch,
      compiler_params=pltpu.CompilerParams(needs_layout_passes=False),
  )


_sc_fused = functools.cache(_sc_fused)

_pre_call = pl.pallas_call(
    _tc_pre,
    out_shape=(
        jax.ShapeDtypeStruct((R, C), jnp.int32),
        jax.ShapeDtypeStruct((R, C), jnp.int32),
        jax.ShapeDtypeStruct((1, 1), jnp.float32),
        jax.ShapeDtypeStruct((1, 1), jnp.int32),
        jax.ShapeDtypeStruct((1, 1), jnp.int32),
    ),
)

_post_call = pl.pallas_call(
    _tc_post,
    out_shape=jax.ShapeDtypeStruct((1, 1), jnp.float32),
)


@jax.jit
def kernel(outputs, targets):
  v = outputs.reshape(R, C)
  t = targets.reshape(R, C)
  key2d, cex2d, possum, npos, nneg = _pre_call(v, t)
  key = key2d.reshape(N)
  cex = cex2d.reshape(N)
  scal16 = jnp.concatenate(
      [npos.reshape(1), nneg.reshape(1),
       jnp.zeros((14,), jnp.int32)]).astype(jnp.int32)
  wq, sg = _sc_fused()(key, cex, scal16)
  out = _post_call(t, wq.reshape(R, C), sg.reshape(R, C), possum, npos, nneg)
  return out.reshape(())
